# baseline (speedup 1.0000x reference)
import functools

import jax
import jax.numpy as jnp
from jax import lax
from jax.experimental import pallas as pl
from jax.experimental.pallas import tpu as pltpu
from jax.experimental.pallas import tpu_sc as plsc

_L = 16


def _sigmoid(x):
    return 1.0 / (1.0 + jnp.exp(-x))


def _mesh():
    return plsc.VectorSubcoreMesh(core_axis_name="c", subcore_axis_name="s")


@functools.lru_cache(maxsize=None)
def _make_exer_kernel(B: int, NC: int, NS: int):
    NW = NC * NS
    assert B % (NW * _L) == 0, (B, NW)
    bpw = B // NW

    @functools.partial(
        pl.kernel,
        mesh=_mesh(),
        out_type=(jax.ShapeDtypeStruct((B,), jnp.float32),
                  jax.ShapeDtypeStruct((B,), jnp.float32)),
        scratch_types=[
            pltpu.VMEM((bpw,), jnp.int32),
            pltpu.VMEM((bpw,), jnp.float32),
            pltpu.VMEM((bpw,), jnp.float32),
            pltpu.VMEM((bpw,), jnp.float32),
            pltpu.VMEM((bpw,), jnp.float32),
            pltpu.SemaphoreType.DMA,
            pltpu.SemaphoreType.DMA,
        ],
    )
    def exer_kernel(exer_id, k_tab, d_tab, f_out, g_out,
                    eidx, krow, drow, fval, gval, sem0, sem1):
        wid = lax.axis_index("s") * NC + lax.axis_index("c")
        base = wid * bpw
        pltpu.sync_copy(exer_id.at[pl.ds(base, bpw)], eidx)
        c0 = pltpu.async_copy(k_tab.at[eidx], krow, sem0)
        c1 = pltpu.async_copy(d_tab.at[eidx], drow, sem1)
        c0.wait()
        c1.wait()

        def body(j, carry):
            sl = pl.ds(j * _L, _L)
            fval[sl] = _sigmoid(drow[sl]) * 10.0
            gval[sl] = _sigmoid(krow[sl])
            return carry

        lax.fori_loop(0, bpw // _L, body, 0, unroll=4)
        pltpu.sync_copy(fval, f_out.at[pl.ds(base, bpw)])
        pltpu.sync_copy(gval, g_out.at[pl.ds(base, bpw)])

    return exer_kernel


@functools.lru_cache(maxsize=None)
def _make_stu_kernel(B: int, NC: int, NS: int):
    NW = NC * NS
    assert B % (NW * _L) == 0, (B, NW)
    bpw = B // NW

    @functools.partial(
        pl.kernel,
        mesh=_mesh(),
        out_type=jax.ShapeDtypeStruct((B,), jnp.float32),
        scratch_types=[
            pltpu.VMEM((bpw,), jnp.int32),
            pltpu.VMEM((bpw,), jnp.float32),
            pltpu.VMEM((bpw,), jnp.float32),
            pltpu.VMEM((bpw,), jnp.float32),
            pltpu.VMEM((bpw,), jnp.float32),
            pltpu.SemaphoreType.DMA,
            pltpu.SemaphoreType.DMA,
        ],
    )
    def stu_kernel(stu_id, s_tab, f_in, g_in, out,
                   sidx, srow, fval, gval, oval, sem0, sem1):
        wid = lax.axis_index("s") * NC + lax.axis_index("c")
        base = wid * bpw
        half = bpw // 2
        lo = pl.ds(0, half)
        hi = pl.ds(half, half)
        pltpu.sync_copy(stu_id.at[pl.ds(base, half)], sidx.at[lo])
        c0 = pltpu.async_copy(s_tab.at[sidx.at[lo]], srow.at[lo], sem0)
        pltpu.sync_copy(stu_id.at[pl.ds(base + half, half)], sidx.at[hi])
        c1 = pltpu.async_copy(s_tab.at[sidx.at[hi]], srow.at[hi], sem1)
        pltpu.sync_copy(f_in.at[pl.ds(base, bpw)], fval)
        pltpu.sync_copy(g_in.at[pl.ds(base, bpw)], gval)

        def body(j, carry):
            sl = pl.ds(j * _L, _L)
            oval[sl] = _sigmoid(fval[sl] * (_sigmoid(srow[sl]) - gval[sl]))
            return carry

        c0.wait()
        lax.fori_loop(0, half // _L, body, 0, unroll=4)
        c1.wait()
        lax.fori_loop(half // _L, bpw // _L, body, 0, unroll=4)
        pltpu.sync_copy(oval, out.at[pl.ds(base, bpw)])

    return stu_kernel


def kernel(stu_id, exer_id, student_emb, k_difficulty, e_discrimination):
    B = stu_id.shape[0]
    info = plsc.get_sparse_core_info()
    exer = _make_exer_kernel(B, info.num_cores, info.num_subcores)
    stu = _make_stu_kernel(B, info.num_cores, info.num_subcores)
    f, g = exer(exer_id.astype(jnp.int32),
                k_difficulty.reshape(-1), e_discrimination.reshape(-1))
    out = stu(stu_id.astype(jnp.int32), student_emb.reshape(-1), f, g)
    return out.reshape(B, 1)


# speedup vs baseline: 1.0109x; 1.0109x over previous
import functools

import jax
import jax.numpy as jnp
from jax import lax
from jax.experimental import pallas as pl
from jax.experimental.pallas import tpu as pltpu
from jax.experimental.pallas import tpu_sc as plsc

_L = 16


def _sigmoid(x):
    return 1.0 / (1.0 + jnp.exp(-x))


def _mesh():
    return plsc.VectorSubcoreMesh(core_axis_name="c", subcore_axis_name="s")


@functools.lru_cache(maxsize=None)
def _make_exer_kernel(B: int, NC: int, NS: int):
    NW = NC * NS
    assert B % (NW * _L) == 0, (B, NW)
    bpw = B // NW

    @functools.partial(
        pl.kernel,
        mesh=_mesh(),
        out_type=(jax.ShapeDtypeStruct((B,), jnp.float32),
                  jax.ShapeDtypeStruct((B,), jnp.float32)),
        scratch_types=[
            pltpu.VMEM((bpw,), jnp.int32),
            pltpu.VMEM((bpw,), jnp.float32),
            pltpu.VMEM((bpw,), jnp.float32),
            pltpu.VMEM((bpw,), jnp.float32),
            pltpu.VMEM((bpw,), jnp.float32),
            pltpu.SemaphoreType.DMA,
            pltpu.SemaphoreType.DMA,
        ],
    )
    def exer_kernel(exer_id, k_tab, d_tab, f_out, g_out,
                    eidx, krow, drow, fval, gval, sem0, sem1):
        wid = lax.axis_index("s") * NC + lax.axis_index("c")
        base = wid * bpw
        pltpu.sync_copy(exer_id.at[pl.ds(base, bpw)], eidx)
        c0 = pltpu.async_copy(k_tab.at[eidx], krow, sem0)
        c1 = pltpu.async_copy(d_tab.at[eidx], drow, sem1)
        c0.wait()
        c1.wait()

        def body(j, carry):
            sl = pl.ds(j * _L, _L)
            fval[sl] = _sigmoid(drow[sl]) * 10.0
            gval[sl] = _sigmoid(krow[sl])
            return carry

        lax.fori_loop(0, bpw // _L, body, 0, unroll=4)
        pltpu.sync_copy(fval, f_out.at[pl.ds(base, bpw)])
        pltpu.sync_copy(gval, g_out.at[pl.ds(base, bpw)])

    return exer_kernel


@functools.lru_cache(maxsize=None)
def _make_stu_kernel(B: int, NC: int, NS: int):
    NW = NC * NS
    assert B % (NW * _L) == 0, (B, NW)
    bpw = B // NW

    @functools.partial(
        pl.kernel,
        mesh=_mesh(),
        out_type=jax.ShapeDtypeStruct((B,), jnp.float32),
        scratch_types=[
            pltpu.VMEM((bpw,), jnp.int32),
            pltpu.VMEM((bpw,), jnp.float32),
            pltpu.VMEM((bpw,), jnp.float32),
            pltpu.VMEM((bpw,), jnp.float32),
            pltpu.VMEM((bpw,), jnp.float32),
            pltpu.SemaphoreType.DMA,
        ],
    )
    def stu_kernel(stu_id, s_tab, f_in, g_in, out,
                   sidx, srow, fval, gval, oval, sem0):
        wid = lax.axis_index("s") * NC + lax.axis_index("c")
        base = wid * bpw
        pltpu.sync_copy(stu_id.at[pl.ds(base, bpw)], sidx)
        c0 = pltpu.async_copy(s_tab.at[sidx], srow, sem0)
        pltpu.sync_copy(f_in.at[pl.ds(base, bpw)], fval)
        pltpu.sync_copy(g_in.at[pl.ds(base, bpw)], gval)
        c0.wait()

        def body(j, carry):
            sl = pl.ds(j * _L, _L)
            oval[sl] = _sigmoid(fval[sl] * (_sigmoid(srow[sl]) - gval[sl]))
            return carry

        lax.fori_loop(0, bpw // _L, body, 0, unroll=4)
        pltpu.sync_copy(oval, out.at[pl.ds(base, bpw)])

    return stu_kernel


def kernel(stu_id, exer_id, student_emb, k_difficulty, e_discrimination):
    B = stu_id.shape[0]
    info = plsc.get_sparse_core_info()
    exer = _make_exer_kernel(B, info.num_cores, info.num_subcores)
    stu = _make_stu_kernel(B, info.num_cores, info.num_subcores)
    f, g = exer(exer_id.astype(jnp.int32),
                k_difficulty.reshape(-1), e_discrimination.reshape(-1))
    out = stu(stu_id.astype(jnp.int32), student_emb.reshape(-1), f, g)
    return out.reshape(B, 1)
